# Initial kernel scaffold; baseline (speedup 1.0000x reference)
import jax
import jax.numpy as jnp
from jax.experimental import pallas as pl


def kernel(x, edge_attr, ne_w, ne_b, bn_gamma, bn_beta, conv_w1, conv_b1, conv_w2, conv_b2, conv_root, conv_bias, wk_w1, wk_b1, wk_w2, wk_b2, uj_w1, uj_b1, uj_w2, uj_b2, zk_w1, zk_b1, zk_w2, zk_b2, edge_index, batch):
    raise NotImplementedError("write your pallas kernel here")


# speedup vs baseline: 1.0990x; 1.0990x over previous
import functools

import jax
import jax.numpy as jnp
from jax import lax
from jax.experimental import pallas as pl
from jax.experimental.pallas import tpu as pltpu

_N = 10000
_E = 160000
_H = 16
_EPS = 1e-5

_EBLK = 2000
_NBLK = 1000


def _embed_body(x_ref, w_ref, b_ref, s_ref, stat_ref):
    i = pl.program_id(0)
    s = jax.nn.relu(
        jnp.dot(x_ref[...], w_ref[...], preferred_element_type=jnp.float32)
        + b_ref[...])
    s_ref[...] = s

    @pl.when(i == 0)
    def _():
        stat_ref[...] = jnp.zeros_like(stat_ref)

    stat_ref[0:1, :] += jnp.sum(s, axis=0, keepdims=True)
    stat_ref[1:2, :] += jnp.sum(s * s, axis=0, keepdims=True)


def _embed(x, ne_w, ne_b):
    nblk = _N // _NBLK
    s, stat = pl.pallas_call(
        _embed_body,
        grid=(nblk,),
        in_specs=[
            pl.BlockSpec((_NBLK, 128), lambda i: (i, 0)),
            pl.BlockSpec((128, _H), lambda i: (0, 0)),
            pl.BlockSpec((1, _H), lambda i: (0, 0)),
        ],
        out_specs=[
            pl.BlockSpec((_NBLK, _H), lambda i: (i, 0)),
            pl.BlockSpec((8, _H), lambda i: (0, 0)),
        ],
        out_shape=[
            jax.ShapeDtypeStruct((_N, _H), jnp.float32),
            jax.ShapeDtypeStruct((8, _H), jnp.float32),
        ],
    )(x, ne_w, ne_b.reshape(1, _H))
    return s, stat


def _norm_body(s_ref, stat_ref, g_ref, be_ref, h_ref):
    mu = stat_ref[0:1, :] / _N
    var = stat_ref[1:2, :] / _N - mu * mu
    scale = g_ref[...] * lax.rsqrt(var + _EPS)
    shift = be_ref[...] - mu * scale
    h_ref[...] = s_ref[...] * scale + shift


def _normalize(s, stat, gamma, beta):
    nblk = _N // _NBLK
    return pl.pallas_call(
        _norm_body,
        grid=(nblk,),
        in_specs=[
            pl.BlockSpec((_NBLK, _H), lambda i: (i, 0)),
            pl.BlockSpec((8, _H), lambda i: (0, 0)),
            pl.BlockSpec((1, _H), lambda i: (0, 0)),
            pl.BlockSpec((1, _H), lambda i: (0, 0)),
        ],
        out_specs=pl.BlockSpec((_NBLK, _H), lambda i: (i, 0)),
        out_shape=jax.ShapeDtypeStruct((_N, _H), jnp.float32),
    )(s, stat, gamma.reshape(1, _H), beta.reshape(1, _H))


def _msg_body(ea_ref, hs_ref, w1_ref, b1_ref, w2_ref, b2_ref, p_ref, sm_ref,
              out_ref):
    a1 = jax.nn.relu(
        jnp.dot(ea_ref[...], w1_ref[...], preferred_element_type=jnp.float32)
        + b1_ref[...])
    we = (jnp.dot(a1, w2_ref[...], preferred_element_type=jnp.float32)
          + b2_ref[...])
    hsrep = jnp.dot(hs_ref[...], p_ref[...], preferred_element_type=jnp.float32)
    q = hsrep * we
    out_ref[...] = jnp.dot(q, sm_ref[...], preferred_element_type=jnp.float32)


def _messages(ea, hs, w1, b1, w2, b2, pmat, smat):
    nblk = _E // _EBLK
    return pl.pallas_call(
        _msg_body,
        grid=(nblk,),
        in_specs=[
            pl.BlockSpec((_EBLK, _H), lambda i: (i, 0)),
            pl.BlockSpec((_EBLK, _H), lambda i: (i, 0)),
            pl.BlockSpec((_H, 2 * _H), lambda i: (0, 0)),
            pl.BlockSpec((1, 2 * _H), lambda i: (0, 0)),
            pl.BlockSpec((2 * _H, _H * _H), lambda i: (0, 0)),
            pl.BlockSpec((1, _H * _H), lambda i: (0, 0)),
            pl.BlockSpec((_H, _H * _H), lambda i: (0, 0)),
            pl.BlockSpec((_H * _H, _H), lambda i: (0, 0)),
        ],
        out_specs=pl.BlockSpec((_EBLK, _H), lambda i: (i, 0)),
        out_shape=jax.ShapeDtypeStruct((_E, _H), jnp.float32),
    )(ea, hs, w1, b1.reshape(1, -1), w2, b2.reshape(1, -1), pmat, smat)


def _upd_body(h_ref, agg_ref, dinv_ref, root_ref, bias_ref, out_ref):
    out_ref[...] = jax.nn.relu(
        jnp.dot(h_ref[...], root_ref[...], preferred_element_type=jnp.float32)
        + agg_ref[...] * dinv_ref[...] + bias_ref[...])


def _update(h, agg, dinv, root, bias):
    nblk = _N // _NBLK
    return pl.pallas_call(
        _upd_body,
        grid=(nblk,),
        in_specs=[
            pl.BlockSpec((_NBLK, _H), lambda i: (i, 0)),
            pl.BlockSpec((_NBLK, _H), lambda i: (i, 0)),
            pl.BlockSpec((_NBLK, _H), lambda i: (i, 0)),
            pl.BlockSpec((_H, _H), lambda i: (0, 0)),
            pl.BlockSpec((1, _H), lambda i: (0, 0)),
        ],
        out_specs=pl.BlockSpec((_NBLK, _H), lambda i: (i, 0)),
        out_shape=jax.ShapeDtypeStruct((_N, _H), jnp.float32),
    )(h, agg, dinv, root, bias.reshape(1, _H))


def kernel(x, edge_attr, ne_w, ne_b, bn_gamma, bn_beta, conv_w1, conv_b1,
           conv_w2, conv_b2, conv_root, conv_bias, wk_w1, wk_b1, wk_w2, wk_b2,
           uj_w1, uj_b1, uj_w2, uj_b2, zk_w1, zk_b1, zk_w2, zk_b2, edge_index,
           batch):
    L = conv_w1.shape[0]
    B = 8
    src = edge_index[0]
    dst = edge_index[1]

    eye = jnp.eye(_H, dtype=jnp.float32)
    pmat = jnp.repeat(eye, _H, axis=1)
    smat = jnp.tile(eye, (_H, 1))

    deg = jax.ops.segment_sum(jnp.ones((_E,), jnp.float32), dst,
                              num_segments=_N)
    dinv = jnp.broadcast_to((1.0 / jnp.clip(deg, 1.0, None))[:, None],
                            (_N, _H))

    s, stat = _embed(x, ne_w, ne_b)
    h = _normalize(s, stat, bn_gamma, bn_beta)

    for l in range(L):
        hs = jnp.take(h, src, axis=0)
        msg = _messages(edge_attr, hs, conv_w1[l], conv_b1[l], conv_w2[l],
                        conv_b2[l], pmat, smat)
        agg = jax.ops.segment_sum(msg, dst, num_segments=_N)
        h = _update(h, agg, dinv, conv_root[l], conv_bias[l])

    gcnt = jnp.clip(
        jax.ops.segment_sum(jnp.ones((_N,), jnp.float32), batch,
                            num_segments=B), 1.0, None)[:, None]
    hg = jax.ops.segment_sum(h, batch, num_segments=B) / gcnt
    wk = (jax.nn.relu(hg @ wk_w1 + wk_b1) @ wk_w2 + wk_b2).squeeze(-1)
    uj = jax.nn.relu(hg @ uj_w1 + uj_b1) @ uj_w2 + uj_b2
    zk = jax.nn.relu(hg @ zk_w1 + zk_b1) @ zk_w2 + zk_b2
    return (wk, uj, zk)


# speedup vs baseline: 1.6009x; 1.4567x over previous
import functools

import jax
import jax.numpy as jnp
from jax import lax
from jax.experimental import pallas as pl
from jax.experimental.pallas import tpu as pltpu
from jax.experimental.pallas import tpu_sc as plsc

_N = 10000
_E = 160000
_H = 16
_EPS = 1e-5

_EBLK = 2000
_NBLK = 1000

_NC = 2
_NS = 16
_NW = _NC * _NS
_PERW = _E // _NW
_GCH = 128
_NFULL = _PERW // _GCH
_TAIL = _PERW - _NFULL * _GCH


def _gather_body(tbl_hbm, idx_hbm, out_hbm, idx_v, rows_v, sem):
    wid = lax.axis_index("s") * _NC + lax.axis_index("c")
    base = wid * _PERW
    pltpu.sync_copy(idx_hbm.at[pl.ds(base, _PERW)], idx_v)

    @pl.loop(0, _NFULL)
    def _fire(j):
        pltpu.make_async_copy(
            tbl_hbm.at[idx_v.at[pl.ds(j * _GCH, _GCH)]],
            rows_v.at[pl.ds(j * _GCH, _GCH)], sem).start()

    pltpu.make_async_copy(
        tbl_hbm.at[idx_v.at[pl.ds(_NFULL * _GCH, _TAIL)]],
        rows_v.at[pl.ds(_NFULL * _GCH, _TAIL)], sem).start()
    pltpu.make_async_copy(tbl_hbm.at[pl.ds(0, _PERW)], rows_v, sem).wait()
    pltpu.sync_copy(rows_v, out_hbm.at[pl.ds(base, _PERW)])


def _gather(tbl, idx):
    return pl.kernel(
        _gather_body,
        out_type=jax.ShapeDtypeStruct((_E, _H), jnp.float32),
        mesh=plsc.VectorSubcoreMesh(core_axis_name="c", subcore_axis_name="s"),
        scratch_types=[
            pltpu.VMEM((_PERW,), jnp.int32),
            pltpu.VMEM((_PERW, _H), jnp.float32),
            pltpu.SemaphoreType.DMA,
        ],
        compiler_params=pltpu.CompilerParams(use_tc_tiling_on_sc=False),
    )(tbl, idx)


def _embed_body(x_ref, w_ref, b_ref, s_ref, stat_ref):
    i = pl.program_id(0)
    s = jax.nn.relu(
        jnp.dot(x_ref[...], w_ref[...], preferred_element_type=jnp.float32)
        + b_ref[...])
    s_ref[...] = s

    @pl.when(i == 0)
    def _():
        stat_ref[...] = jnp.zeros_like(stat_ref)

    stat_ref[0:1, :] += jnp.sum(s, axis=0, keepdims=True)
    stat_ref[1:2, :] += jnp.sum(s * s, axis=0, keepdims=True)


def _embed(x, ne_w, ne_b):
    nblk = _N // _NBLK
    s, stat = pl.pallas_call(
        _embed_body,
        grid=(nblk,),
        in_specs=[
            pl.BlockSpec((_NBLK, 128), lambda i: (i, 0)),
            pl.BlockSpec((128, _H), lambda i: (0, 0)),
            pl.BlockSpec((1, _H), lambda i: (0, 0)),
        ],
        out_specs=[
            pl.BlockSpec((_NBLK, _H), lambda i: (i, 0)),
            pl.BlockSpec((8, _H), lambda i: (0, 0)),
        ],
        out_shape=[
            jax.ShapeDtypeStruct((_N, _H), jnp.float32),
            jax.ShapeDtypeStruct((8, _H), jnp.float32),
        ],
    )(x, ne_w, ne_b.reshape(1, _H))
    return s, stat


def _norm_body(s_ref, stat_ref, g_ref, be_ref, h_ref):
    mu = stat_ref[0:1, :] / _N
    var = stat_ref[1:2, :] / _N - mu * mu
    scale = g_ref[...] * lax.rsqrt(var + _EPS)
    shift = be_ref[...] - mu * scale
    h_ref[...] = s_ref[...] * scale + shift


def _normalize(s, stat, gamma, beta):
    nblk = _N // _NBLK
    return pl.pallas_call(
        _norm_body,
        grid=(nblk,),
        in_specs=[
            pl.BlockSpec((_NBLK, _H), lambda i: (i, 0)),
            pl.BlockSpec((8, _H), lambda i: (0, 0)),
            pl.BlockSpec((1, _H), lambda i: (0, 0)),
            pl.BlockSpec((1, _H), lambda i: (0, 0)),
        ],
        out_specs=pl.BlockSpec((_NBLK, _H), lambda i: (i, 0)),
        out_shape=jax.ShapeDtypeStruct((_N, _H), jnp.float32),
    )(s, stat, gamma.reshape(1, _H), beta.reshape(1, _H))


def _msg_body(ea_ref, hs_ref, w1_ref, b1_ref, w2_ref, b2_ref, p_ref, sm_ref,
              out_ref):
    a1 = jax.nn.relu(
        jnp.dot(ea_ref[...], w1_ref[...], preferred_element_type=jnp.float32)
        + b1_ref[...])
    we = (jnp.dot(a1, w2_ref[...], preferred_element_type=jnp.float32)
          + b2_ref[...])
    hsrep = jnp.dot(hs_ref[...], p_ref[...], preferred_element_type=jnp.float32)
    q = hsrep * we
    out_ref[...] = jnp.dot(q, sm_ref[...], preferred_element_type=jnp.float32)


def _messages(ea, hs, w1, b1, w2, b2, pmat, smat):
    nblk = _E // _EBLK
    return pl.pallas_call(
        _msg_body,
        grid=(nblk,),
        in_specs=[
            pl.BlockSpec((_EBLK, _H), lambda i: (i, 0)),
            pl.BlockSpec((_EBLK, _H), lambda i: (i, 0)),
            pl.BlockSpec((_H, 2 * _H), lambda i: (0, 0)),
            pl.BlockSpec((1, 2 * _H), lambda i: (0, 0)),
            pl.BlockSpec((2 * _H, _H * _H), lambda i: (0, 0)),
            pl.BlockSpec((1, _H * _H), lambda i: (0, 0)),
            pl.BlockSpec((_H, _H * _H), lambda i: (0, 0)),
            pl.BlockSpec((_H * _H, _H), lambda i: (0, 0)),
        ],
        out_specs=pl.BlockSpec((_EBLK, _H), lambda i: (i, 0)),
        out_shape=jax.ShapeDtypeStruct((_E, _H), jnp.float32),
    )(ea, hs, w1, b1.reshape(1, -1), w2, b2.reshape(1, -1), pmat, smat)


def _upd_body(h_ref, agg_ref, dinv_ref, root_ref, bias_ref, out_ref):
    out_ref[...] = jax.nn.relu(
        jnp.dot(h_ref[...], root_ref[...], preferred_element_type=jnp.float32)
        + agg_ref[...] * dinv_ref[...] + bias_ref[...])


def _update(h, agg, dinv, root, bias):
    nblk = _N // _NBLK
    return pl.pallas_call(
        _upd_body,
        grid=(nblk,),
        in_specs=[
            pl.BlockSpec((_NBLK, _H), lambda i: (i, 0)),
            pl.BlockSpec((_NBLK, _H), lambda i: (i, 0)),
            pl.BlockSpec((_NBLK, _H), lambda i: (i, 0)),
            pl.BlockSpec((_H, _H), lambda i: (0, 0)),
            pl.BlockSpec((1, _H), lambda i: (0, 0)),
        ],
        out_specs=pl.BlockSpec((_NBLK, _H), lambda i: (i, 0)),
        out_shape=jax.ShapeDtypeStruct((_N, _H), jnp.float32),
    )(h, agg, dinv, root, bias.reshape(1, _H))


def kernel(x, edge_attr, ne_w, ne_b, bn_gamma, bn_beta, conv_w1, conv_b1,
           conv_w2, conv_b2, conv_root, conv_bias, wk_w1, wk_b1, wk_w2, wk_b2,
           uj_w1, uj_b1, uj_w2, uj_b2, zk_w1, zk_b1, zk_w2, zk_b2, edge_index,
           batch):
    L = conv_w1.shape[0]
    B = 8
    src = edge_index[0]
    dst = edge_index[1]

    eye = jnp.eye(_H, dtype=jnp.float32)
    pmat = jnp.repeat(eye, _H, axis=1)
    smat = jnp.tile(eye, (_H, 1))

    deg = jax.ops.segment_sum(jnp.ones((_E,), jnp.float32), dst,
                              num_segments=_N)
    dinv = jnp.broadcast_to((1.0 / jnp.clip(deg, 1.0, None))[:, None],
                            (_N, _H))

    s, stat = _embed(x, ne_w, ne_b)
    h = _normalize(s, stat, bn_gamma, bn_beta)

    for l in range(L):
        hs = _gather(h, src)
        msg = _messages(edge_attr, hs, conv_w1[l], conv_b1[l], conv_w2[l],
                        conv_b2[l], pmat, smat)
        agg = jax.ops.segment_sum(msg, dst, num_segments=_N)
        h = _update(h, agg, dinv, conv_root[l], conv_bias[l])

    gcnt = jnp.clip(
        jax.ops.segment_sum(jnp.ones((_N,), jnp.float32), batch,
                            num_segments=B), 1.0, None)[:, None]
    hg = jax.ops.segment_sum(h, batch, num_segments=B) / gcnt
    wk = (jax.nn.relu(hg @ wk_w1 + wk_b1) @ wk_w2 + wk_b2).squeeze(-1)
    uj = jax.nn.relu(hg @ uj_w1 + uj_b1) @ uj_w2 + uj_b2
    zk = jax.nn.relu(hg @ zk_w1 + zk_b1) @ zk_w2 + zk_b2
    return (wk, uj, zk)


# speedup vs baseline: 3.2354x; 2.0210x over previous
import functools

import jax
import jax.numpy as jnp
from jax import lax
from jax.experimental import pallas as pl
from jax.experimental.pallas import tpu as pltpu
from jax.experimental.pallas import tpu_sc as plsc

_N = 10000
_E = 160000
_H = 16
_EPS = 1e-5

_EBLK = 2000
_NBLK = 1000

_NC = 2
_NS = 16
_NW = _NC * _NS
_PERW = _E // _NW
_GCH = 128
_NFULL = _PERW // _GCH
_TAIL = _PERW - _NFULL * _GCH


def _gather_body(tbl_hbm, idx_hbm, out_hbm, idx_v, rows_v, sem):
    wid = lax.axis_index("s") * _NC + lax.axis_index("c")
    base = wid * _PERW
    pltpu.sync_copy(idx_hbm.at[pl.ds(base, _PERW)], idx_v)

    @pl.loop(0, _NFULL)
    def _fire(j):
        pltpu.make_async_copy(
            tbl_hbm.at[idx_v.at[pl.ds(j * _GCH, _GCH)]],
            rows_v.at[pl.ds(j * _GCH, _GCH)], sem).start()

    pltpu.make_async_copy(
        tbl_hbm.at[idx_v.at[pl.ds(_NFULL * _GCH, _TAIL)]],
        rows_v.at[pl.ds(_NFULL * _GCH, _TAIL)], sem).start()
    pltpu.make_async_copy(tbl_hbm.at[pl.ds(0, _PERW)], rows_v, sem).wait()
    pltpu.sync_copy(rows_v, out_hbm.at[pl.ds(base, _PERW)])


def _gather(tbl, idx):
    return pl.kernel(
        _gather_body,
        out_type=jax.ShapeDtypeStruct((_E, _H), jnp.float32),
        mesh=plsc.VectorSubcoreMesh(core_axis_name="c", subcore_axis_name="s"),
        scratch_types=[
            pltpu.VMEM((_PERW,), jnp.int32),
            pltpu.VMEM((_PERW, _H), jnp.float32),
            pltpu.SemaphoreType.DMA,
        ],
        compiler_params=pltpu.CompilerParams(use_tc_tiling_on_sc=False),
    )(tbl, idx)


_CCH = 125
_NCHK = _PERW // _CCH
_NSTRIPE = _N // _NS


def _scatter_body(msg_hbm, dst_hbm, zeros_hbm, out_hbm, idx_v, rows_v, agg_sh):
    c = lax.axis_index("c")
    s = lax.axis_index("s")
    wid = s * _NC + c
    base = wid * _PERW
    pltpu.sync_copy(zeros_hbm.at[pl.ds(s * _NSTRIPE, _NSTRIPE)],
                    agg_sh.at[pl.ds(s * _NSTRIPE, _NSTRIPE)])
    pltpu.sync_copy(dst_hbm.at[wid], idx_v)
    pltpu.sync_copy(msg_hbm.at[pl.ds(base, _PERW)], rows_v)
    plsc.subcore_barrier()

    @pl.loop(0, _NCHK)
    def _sc(j):
        pltpu.sync_copy(rows_v.at[pl.ds(j * _CCH, _CCH)],
                        agg_sh.at[idx_v.at[j]], add=True)

    plsc.subcore_barrier()
    pltpu.sync_copy(agg_sh.at[pl.ds(s * _NSTRIPE, _NSTRIPE)],
                    out_hbm.at[c, pl.ds(s * _NSTRIPE, _NSTRIPE)])


def _scatter(msg, dst3d, zeros_n):
    return pl.kernel(
        _scatter_body,
        out_type=jax.ShapeDtypeStruct((_NC, _N, _H), jnp.float32),
        mesh=plsc.VectorSubcoreMesh(core_axis_name="c", subcore_axis_name="s"),
        scratch_types=[
            pltpu.VMEM((_NCHK, _CCH), jnp.int32),
            pltpu.VMEM((_PERW, _H), jnp.float32),
            pltpu.VMEM_SHARED((_N, _H), jnp.float32),
        ],
        compiler_params=pltpu.CompilerParams(use_tc_tiling_on_sc=False),
    )(msg, dst3d, zeros_n)


def _deg_body(dst_hbm, zeros_hbm, ones_hbm, out_hbm, idx_v, ones_v, deg_sh):
    c = lax.axis_index("c")
    s = lax.axis_index("s")
    wid = s * _NC + c
    pltpu.sync_copy(zeros_hbm.at[pl.ds(s * _NSTRIPE, _NSTRIPE)],
                    deg_sh.at[pl.ds(s * _NSTRIPE, _NSTRIPE)])
    pltpu.sync_copy(dst_hbm.at[wid], idx_v)
    pltpu.sync_copy(ones_hbm, ones_v)
    plsc.subcore_barrier()

    @pl.loop(0, _NCHK)
    def _sc(j):
        pltpu.sync_copy(ones_v, deg_sh.at[idx_v.at[j]], add=True)

    plsc.subcore_barrier()
    pltpu.sync_copy(deg_sh.at[pl.ds(s * _NSTRIPE, _NSTRIPE)],
                    out_hbm.at[c, pl.ds(s * _NSTRIPE, _NSTRIPE)])


def _degrees(dst3d, zeros_n, ones_c):
    return pl.kernel(
        _deg_body,
        out_type=jax.ShapeDtypeStruct((_NC, _N, _H), jnp.float32),
        mesh=plsc.VectorSubcoreMesh(core_axis_name="c", subcore_axis_name="s"),
        scratch_types=[
            pltpu.VMEM((_NCHK, _CCH), jnp.int32),
            pltpu.VMEM((_CCH, _H), jnp.float32),
            pltpu.VMEM_SHARED((_N, _H), jnp.float32),
        ],
        compiler_params=pltpu.CompilerParams(use_tc_tiling_on_sc=False),
    )(dst3d, zeros_n, ones_c)


def _embed_body(x_ref, w_ref, b_ref, s_ref, stat_ref):
    i = pl.program_id(0)
    s = jax.nn.relu(
        jnp.dot(x_ref[...], w_ref[...], preferred_element_type=jnp.float32)
        + b_ref[...])
    s_ref[...] = s

    @pl.when(i == 0)
    def _():
        stat_ref[...] = jnp.zeros_like(stat_ref)

    stat_ref[0:1, :] += jnp.sum(s, axis=0, keepdims=True)
    stat_ref[1:2, :] += jnp.sum(s * s, axis=0, keepdims=True)


def _embed(x, ne_w, ne_b):
    nblk = _N // _NBLK
    s, stat = pl.pallas_call(
        _embed_body,
        grid=(nblk,),
        in_specs=[
            pl.BlockSpec((_NBLK, 128), lambda i: (i, 0)),
            pl.BlockSpec((128, _H), lambda i: (0, 0)),
            pl.BlockSpec((1, _H), lambda i: (0, 0)),
        ],
        out_specs=[
            pl.BlockSpec((_NBLK, _H), lambda i: (i, 0)),
            pl.BlockSpec((8, _H), lambda i: (0, 0)),
        ],
        out_shape=[
            jax.ShapeDtypeStruct((_N, _H), jnp.float32),
            jax.ShapeDtypeStruct((8, _H), jnp.float32),
        ],
    )(x, ne_w, ne_b.reshape(1, _H))
    return s, stat


def _norm_body(s_ref, stat_ref, g_ref, be_ref, h_ref):
    mu = stat_ref[0:1, :] / _N
    var = stat_ref[1:2, :] / _N - mu * mu
    scale = g_ref[...] * lax.rsqrt(var + _EPS)
    shift = be_ref[...] - mu * scale
    h_ref[...] = s_ref[...] * scale + shift


def _normalize(s, stat, gamma, beta):
    nblk = _N // _NBLK
    return pl.pallas_call(
        _norm_body,
        grid=(nblk,),
        in_specs=[
            pl.BlockSpec((_NBLK, _H), lambda i: (i, 0)),
            pl.BlockSpec((8, _H), lambda i: (0, 0)),
            pl.BlockSpec((1, _H), lambda i: (0, 0)),
            pl.BlockSpec((1, _H), lambda i: (0, 0)),
        ],
        out_specs=pl.BlockSpec((_NBLK, _H), lambda i: (i, 0)),
        out_shape=jax.ShapeDtypeStruct((_N, _H), jnp.float32),
    )(s, stat, gamma.reshape(1, _H), beta.reshape(1, _H))


def _msg_body(ea_ref, hs_ref, w1_ref, b1_ref, w2_ref, b2_ref, p_ref, sm_ref,
              out_ref):
    a1 = jax.nn.relu(
        jnp.dot(ea_ref[...], w1_ref[...], preferred_element_type=jnp.float32)
        + b1_ref[...])
    we = (jnp.dot(a1, w2_ref[...], preferred_element_type=jnp.float32)
          + b2_ref[...])
    hsrep = jnp.dot(hs_ref[...], p_ref[...], preferred_element_type=jnp.float32)
    q = hsrep * we
    out_ref[...] = jnp.dot(q, sm_ref[...], preferred_element_type=jnp.float32)


def _messages(ea, hs, w1, b1, w2, b2, pmat, smat):
    nblk = _E // _EBLK
    return pl.pallas_call(
        _msg_body,
        grid=(nblk,),
        in_specs=[
            pl.BlockSpec((_EBLK, _H), lambda i: (i, 0)),
            pl.BlockSpec((_EBLK, _H), lambda i: (i, 0)),
            pl.BlockSpec((_H, 2 * _H), lambda i: (0, 0)),
            pl.BlockSpec((1, 2 * _H), lambda i: (0, 0)),
            pl.BlockSpec((2 * _H, _H * _H), lambda i: (0, 0)),
            pl.BlockSpec((1, _H * _H), lambda i: (0, 0)),
            pl.BlockSpec((_H, _H * _H), lambda i: (0, 0)),
            pl.BlockSpec((_H * _H, _H), lambda i: (0, 0)),
        ],
        out_specs=pl.BlockSpec((_EBLK, _H), lambda i: (i, 0)),
        out_shape=jax.ShapeDtypeStruct((_E, _H), jnp.float32),
    )(ea, hs, w1, b1.reshape(1, -1), w2, b2.reshape(1, -1), pmat, smat)


def _upd_body(h_ref, a0_ref, a1_ref, dinv_ref, root_ref, bias_ref, out_ref):
    out_ref[...] = jax.nn.relu(
        jnp.dot(h_ref[...], root_ref[...], preferred_element_type=jnp.float32)
        + (a0_ref[...] + a1_ref[...]) * dinv_ref[...] + bias_ref[...])


def _update(h, aggp, dinv, root, bias):
    nblk = _N // _NBLK
    return pl.pallas_call(
        _upd_body,
        grid=(nblk,),
        in_specs=[
            pl.BlockSpec((_NBLK, _H), lambda i: (i, 0)),
            pl.BlockSpec((_NBLK, _H), lambda i: (i, 0)),
            pl.BlockSpec((_NBLK, _H), lambda i: (i, 0)),
            pl.BlockSpec((_NBLK, _H), lambda i: (i, 0)),
            pl.BlockSpec((_H, _H), lambda i: (0, 0)),
            pl.BlockSpec((1, _H), lambda i: (0, 0)),
        ],
        out_specs=pl.BlockSpec((_NBLK, _H), lambda i: (i, 0)),
        out_shape=jax.ShapeDtypeStruct((_N, _H), jnp.float32),
    )(h, aggp[0], aggp[1], dinv, root, bias.reshape(1, _H))


def _dinv_body(d0_ref, d1_ref, out_ref):
    out_ref[...] = 1.0 / jnp.clip(d0_ref[...] + d1_ref[...], 1.0, None)


def _dinv(degp):
    nblk = _N // _NBLK
    return pl.pallas_call(
        _dinv_body,
        grid=(nblk,),
        in_specs=[
            pl.BlockSpec((_NBLK, _H), lambda i: (i, 0)),
            pl.BlockSpec((_NBLK, _H), lambda i: (i, 0)),
        ],
        out_specs=pl.BlockSpec((_NBLK, _H), lambda i: (i, 0)),
        out_shape=jax.ShapeDtypeStruct((_N, _H), jnp.float32),
    )(degp[0], degp[1])


def kernel(x, edge_attr, ne_w, ne_b, bn_gamma, bn_beta, conv_w1, conv_b1,
           conv_w2, conv_b2, conv_root, conv_bias, wk_w1, wk_b1, wk_w2, wk_b2,
           uj_w1, uj_b1, uj_w2, uj_b2, zk_w1, zk_b1, zk_w2, zk_b2, edge_index,
           batch):
    L = conv_w1.shape[0]
    B = 8
    src = edge_index[0]
    dst = edge_index[1]

    eye = jnp.eye(_H, dtype=jnp.float32)
    pmat = jnp.repeat(eye, _H, axis=1)
    smat = jnp.tile(eye, (_H, 1))

    dst3d = dst.reshape(_NW, _NCHK, _CCH)
    zeros_n = jnp.zeros((_N, _H), jnp.float32)
    ones_c = jnp.ones((_CCH, _H), jnp.float32)

    degp = _degrees(dst3d, zeros_n, ones_c)
    dinv = _dinv(degp)

    s, stat = _embed(x, ne_w, ne_b)
    h = _normalize(s, stat, bn_gamma, bn_beta)

    for l in range(L):
        hs = _gather(h, src)
        msg = _messages(edge_attr, hs, conv_w1[l], conv_b1[l], conv_w2[l],
                        conv_b2[l], pmat, smat)
        aggp = _scatter(msg, dst3d, zeros_n)
        h = _update(h, aggp, dinv, conv_root[l], conv_bias[l])

    gcnt = jnp.clip(
        jax.ops.segment_sum(jnp.ones((_N,), jnp.float32), batch,
                            num_segments=B), 1.0, None)[:, None]
    hg = jax.ops.segment_sum(h, batch, num_segments=B) / gcnt
    wk = (jax.nn.relu(hg @ wk_w1 + wk_b1) @ wk_w2 + wk_b2).squeeze(-1)
    uj = jax.nn.relu(hg @ uj_w1 + uj_b1) @ uj_w2 + uj_b2
    zk = jax.nn.relu(hg @ zk_w1 + zk_b1) @ zk_w2 + zk_b2
    return (wk, uj, zk)


# speedup vs baseline: 3.2949x; 1.0184x over previous
import functools

import jax
import jax.numpy as jnp
from jax import lax
from jax.experimental import pallas as pl
from jax.experimental.pallas import tpu as pltpu
from jax.experimental.pallas import tpu_sc as plsc

_N = 10000
_E = 160000
_H = 16
_EPS = 1e-5

_EBLK = 2000
_NBLK = 1000

_NC = 2
_NS = 16
_NW = _NC * _NS
_PERW = _E // _NW
_GCH = 128
_NFULL = _PERW // _GCH
_TAIL = _PERW - _NFULL * _GCH


def _gather_body(tbl_hbm, idx_hbm, out_hbm, idx_v, rows_v, sem):
    wid = lax.axis_index("s") * _NC + lax.axis_index("c")
    base = wid * _PERW
    pltpu.sync_copy(idx_hbm.at[pl.ds(base, _PERW)], idx_v)

    @pl.loop(0, _NFULL)
    def _fire(j):
        pltpu.make_async_copy(
            tbl_hbm.at[idx_v.at[pl.ds(j * _GCH, _GCH)]],
            rows_v.at[pl.ds(j * _GCH, _GCH)], sem).start()

    pltpu.make_async_copy(
        tbl_hbm.at[idx_v.at[pl.ds(_NFULL * _GCH, _TAIL)]],
        rows_v.at[pl.ds(_NFULL * _GCH, _TAIL)], sem).start()
    pltpu.make_async_copy(tbl_hbm.at[pl.ds(0, _PERW)], rows_v, sem).wait()
    pltpu.sync_copy(rows_v, out_hbm.at[pl.ds(base, _PERW)])


def _gather(tbl, idx):
    return pl.kernel(
        _gather_body,
        out_type=jax.ShapeDtypeStruct((_E, _H), jnp.float32),
        mesh=plsc.VectorSubcoreMesh(core_axis_name="c", subcore_axis_name="s"),
        scratch_types=[
            pltpu.VMEM((_PERW,), jnp.int32),
            pltpu.VMEM((_PERW, _H), jnp.float32),
            pltpu.SemaphoreType.DMA,
        ],
        compiler_params=pltpu.CompilerParams(use_tc_tiling_on_sc=False),
    )(tbl, idx)


_CCH = 125
_NCHK = _PERW // _CCH
_NSTRIPE = _N // _NS


def _scatter_body(msg_hbm, dst_hbm, zeros_hbm, out_hbm, idx_v, rows_v, agg_sh):
    c = lax.axis_index("c")
    s = lax.axis_index("s")
    wid = s * _NC + c
    base = wid * _PERW
    pltpu.sync_copy(zeros_hbm.at[pl.ds(s * _NSTRIPE, _NSTRIPE)],
                    agg_sh.at[pl.ds(s * _NSTRIPE, _NSTRIPE)])
    pltpu.sync_copy(dst_hbm.at[wid], idx_v)
    pltpu.sync_copy(msg_hbm.at[pl.ds(base, _PERW)], rows_v)
    plsc.subcore_barrier()

    @pl.loop(0, _NCHK)
    def _sc(j):
        pltpu.sync_copy(rows_v.at[pl.ds(j * _CCH, _CCH)],
                        agg_sh.at[idx_v.at[j]], add=True)

    plsc.subcore_barrier()
    pltpu.sync_copy(agg_sh.at[pl.ds(s * _NSTRIPE, _NSTRIPE)],
                    out_hbm.at[c, pl.ds(s * _NSTRIPE, _NSTRIPE)])


def _scatter(msg, dst3d, zeros_n):
    return pl.kernel(
        _scatter_body,
        out_type=jax.ShapeDtypeStruct((_NC, _N, _H), jnp.float32),
        mesh=plsc.VectorSubcoreMesh(core_axis_name="c", subcore_axis_name="s"),
        scratch_types=[
            pltpu.VMEM((_NCHK, _CCH), jnp.int32),
            pltpu.VMEM((_PERW, _H), jnp.float32),
            pltpu.VMEM_SHARED((_N, _H), jnp.float32),
        ],
        compiler_params=pltpu.CompilerParams(use_tc_tiling_on_sc=False),
    )(msg, dst3d, zeros_n)


def _deg_body(dst_hbm, zeros_hbm, ones_hbm, out_hbm, idx_v, ones_v, deg_sh):
    c = lax.axis_index("c")
    s = lax.axis_index("s")
    wid = s * _NC + c
    pltpu.sync_copy(zeros_hbm.at[pl.ds(s * _NSTRIPE, _NSTRIPE)],
                    deg_sh.at[pl.ds(s * _NSTRIPE, _NSTRIPE)])
    pltpu.sync_copy(dst_hbm.at[wid], idx_v)
    pltpu.sync_copy(ones_hbm, ones_v)
    plsc.subcore_barrier()

    @pl.loop(0, _NCHK)
    def _sc(j):
        pltpu.sync_copy(ones_v, deg_sh.at[idx_v.at[j]], add=True)

    plsc.subcore_barrier()
    pltpu.sync_copy(deg_sh.at[pl.ds(s * _NSTRIPE, _NSTRIPE)],
                    out_hbm.at[c, pl.ds(s * _NSTRIPE, _NSTRIPE)])


def _degrees(dst3d, zeros_n, ones_c):
    return pl.kernel(
        _deg_body,
        out_type=jax.ShapeDtypeStruct((_NC, _N, _H), jnp.float32),
        mesh=plsc.VectorSubcoreMesh(core_axis_name="c", subcore_axis_name="s"),
        scratch_types=[
            pltpu.VMEM((_NCHK, _CCH), jnp.int32),
            pltpu.VMEM((_CCH, _H), jnp.float32),
            pltpu.VMEM_SHARED((_N, _H), jnp.float32),
        ],
        compiler_params=pltpu.CompilerParams(use_tc_tiling_on_sc=False),
    )(dst3d, zeros_n, ones_c)


def _embed_body(x_ref, w_ref, b_ref, s_ref, stat_ref):
    i = pl.program_id(0)
    s = jax.nn.relu(
        jnp.dot(x_ref[...], w_ref[...], preferred_element_type=jnp.float32)
        + b_ref[...])
    s_ref[...] = s

    @pl.when(i == 0)
    def _():
        stat_ref[...] = jnp.zeros_like(stat_ref)

    stat_ref[0:1, :] += jnp.sum(s, axis=0, keepdims=True)
    stat_ref[1:2, :] += jnp.sum(s * s, axis=0, keepdims=True)


def _embed(x, ne_w, ne_b):
    nblk = _N // _NBLK
    s, stat = pl.pallas_call(
        _embed_body,
        grid=(nblk,),
        in_specs=[
            pl.BlockSpec((_NBLK, 128), lambda i: (i, 0)),
            pl.BlockSpec((128, _H), lambda i: (0, 0)),
            pl.BlockSpec((1, _H), lambda i: (0, 0)),
        ],
        out_specs=[
            pl.BlockSpec((_NBLK, _H), lambda i: (i, 0)),
            pl.BlockSpec((8, _H), lambda i: (0, 0)),
        ],
        out_shape=[
            jax.ShapeDtypeStruct((_N, _H), jnp.float32),
            jax.ShapeDtypeStruct((8, _H), jnp.float32),
        ],
    )(x, ne_w, ne_b.reshape(1, _H))
    return s, stat


def _norm_body(s_ref, stat_ref, g_ref, be_ref, h_ref):
    mu = stat_ref[0:1, :] / _N
    var = stat_ref[1:2, :] / _N - mu * mu
    scale = g_ref[...] * lax.rsqrt(var + _EPS)
    shift = be_ref[...] - mu * scale
    h_ref[...] = s_ref[...] * scale + shift


def _normalize(s, stat, gamma, beta):
    nblk = _N // _NBLK
    return pl.pallas_call(
        _norm_body,
        grid=(nblk,),
        in_specs=[
            pl.BlockSpec((_NBLK, _H), lambda i: (i, 0)),
            pl.BlockSpec((8, _H), lambda i: (0, 0)),
            pl.BlockSpec((1, _H), lambda i: (0, 0)),
            pl.BlockSpec((1, _H), lambda i: (0, 0)),
        ],
        out_specs=pl.BlockSpec((_NBLK, _H), lambda i: (i, 0)),
        out_shape=jax.ShapeDtypeStruct((_N, _H), jnp.float32),
    )(s, stat, gamma.reshape(1, _H), beta.reshape(1, _H))


def _msg_body(ea_ref, hs_ref, w1_ref, b1_ref, w2_ref, b2_ref, p_ref, sm_ref,
              out_ref):
    a1 = jax.nn.relu(
        jnp.dot(ea_ref[...], w1_ref[...], preferred_element_type=jnp.float32)
        + b1_ref[...])
    we = (jnp.dot(a1.astype(jnp.bfloat16), w2_ref[...],
                  preferred_element_type=jnp.float32) + b2_ref[...])
    hsrep = jnp.dot(hs_ref[...].astype(jnp.bfloat16), p_ref[...],
                    preferred_element_type=jnp.float32)
    q = (hsrep * we).astype(jnp.bfloat16)
    out_ref[...] = jnp.dot(q, sm_ref[...], preferred_element_type=jnp.float32)


def _messages(ea, hs, w1, b1, w2, b2, pmat, smat):
    nblk = _E // _EBLK
    return pl.pallas_call(
        _msg_body,
        grid=(nblk,),
        in_specs=[
            pl.BlockSpec((_EBLK, _H), lambda i: (i, 0)),
            pl.BlockSpec((_EBLK, _H), lambda i: (i, 0)),
            pl.BlockSpec((_H, 2 * _H), lambda i: (0, 0)),
            pl.BlockSpec((1, 2 * _H), lambda i: (0, 0)),
            pl.BlockSpec((2 * _H, _H * _H), lambda i: (0, 0)),
            pl.BlockSpec((1, _H * _H), lambda i: (0, 0)),
            pl.BlockSpec((_H, _H * _H), lambda i: (0, 0)),
            pl.BlockSpec((_H * _H, _H), lambda i: (0, 0)),
        ],
        out_specs=pl.BlockSpec((_EBLK, _H), lambda i: (i, 0)),
        out_shape=jax.ShapeDtypeStruct((_E, _H), jnp.float32),
    )(ea, hs, w1, b1.reshape(1, -1).astype(jnp.float32),
      w2, b2.reshape(1, -1).astype(jnp.float32), pmat, smat)


def _upd_body(h_ref, a0_ref, a1_ref, dinv_ref, root_ref, bias_ref, out_ref):
    out_ref[...] = jax.nn.relu(
        jnp.dot(h_ref[...], root_ref[...], preferred_element_type=jnp.float32)
        + (a0_ref[...] + a1_ref[...]) * dinv_ref[...] + bias_ref[...])


def _update(h, aggp, dinv, root, bias):
    nblk = _N // _NBLK
    return pl.pallas_call(
        _upd_body,
        grid=(nblk,),
        in_specs=[
            pl.BlockSpec((_NBLK, _H), lambda i: (i, 0)),
            pl.BlockSpec((_NBLK, _H), lambda i: (i, 0)),
            pl.BlockSpec((_NBLK, _H), lambda i: (i, 0)),
            pl.BlockSpec((_NBLK, _H), lambda i: (i, 0)),
            pl.BlockSpec((_H, _H), lambda i: (0, 0)),
            pl.BlockSpec((1, _H), lambda i: (0, 0)),
        ],
        out_specs=pl.BlockSpec((_NBLK, _H), lambda i: (i, 0)),
        out_shape=jax.ShapeDtypeStruct((_N, _H), jnp.float32),
    )(h, aggp[0], aggp[1], dinv, root, bias.reshape(1, _H))


def _dinv_body(d0_ref, d1_ref, out_ref):
    out_ref[...] = 1.0 / jnp.clip(d0_ref[...] + d1_ref[...], 1.0, None)


def _dinv(degp):
    nblk = _N // _NBLK
    return pl.pallas_call(
        _dinv_body,
        grid=(nblk,),
        in_specs=[
            pl.BlockSpec((_NBLK, _H), lambda i: (i, 0)),
            pl.BlockSpec((_NBLK, _H), lambda i: (i, 0)),
        ],
        out_specs=pl.BlockSpec((_NBLK, _H), lambda i: (i, 0)),
        out_shape=jax.ShapeDtypeStruct((_N, _H), jnp.float32),
    )(degp[0], degp[1])


def kernel(x, edge_attr, ne_w, ne_b, bn_gamma, bn_beta, conv_w1, conv_b1,
           conv_w2, conv_b2, conv_root, conv_bias, wk_w1, wk_b1, wk_w2, wk_b2,
           uj_w1, uj_b1, uj_w2, uj_b2, zk_w1, zk_b1, zk_w2, zk_b2, edge_index,
           batch):
    L = conv_w1.shape[0]
    B = 8
    src = edge_index[0]
    dst = edge_index[1]

    eye = jnp.eye(_H, dtype=jnp.bfloat16)
    pmat = jnp.repeat(eye, _H, axis=1)
    smat = jnp.tile(eye, (_H, 1))
    ea_bf = edge_attr.astype(jnp.bfloat16)

    dst3d = dst.reshape(_NW, _NCHK, _CCH)
    zeros_n = jnp.zeros((_N, _H), jnp.float32)
    ones_c = jnp.ones((_CCH, _H), jnp.float32)

    degp = _degrees(dst3d, zeros_n, ones_c)
    dinv = _dinv(degp)

    s, stat = _embed(x, ne_w, ne_b)
    h = _normalize(s, stat, bn_gamma, bn_beta)

    for l in range(L):
        hs = _gather(h, src)
        msg = _messages(ea_bf, hs, conv_w1[l].astype(jnp.bfloat16),
                        conv_b1[l], conv_w2[l].astype(jnp.bfloat16),
                        conv_b2[l], pmat, smat)
        aggp = _scatter(msg, dst3d, zeros_n)
        h = _update(h, aggp, dinv, conv_root[l], conv_bias[l])

    gcnt = jnp.clip(
        jax.ops.segment_sum(jnp.ones((_N,), jnp.float32), batch,
                            num_segments=B), 1.0, None)[:, None]
    hg = jax.ops.segment_sum(h, batch, num_segments=B) / gcnt
    wk = (jax.nn.relu(hg @ wk_w1 + wk_b1) @ wk_w2 + wk_b2).squeeze(-1)
    uj = jax.nn.relu(hg @ uj_w1 + uj_b1) @ uj_w2 + uj_b2
    zk = jax.nn.relu(hg @ zk_w1 + zk_b1) @ zk_w2 + zk_b2
    return (wk, uj, zk)


# speedup vs baseline: 3.7537x; 1.1392x over previous
import functools

import jax
import jax.numpy as jnp
from jax import lax
from jax.experimental import pallas as pl
from jax.experimental.pallas import tpu as pltpu
from jax.experimental.pallas import tpu_sc as plsc

_N = 10000
_E = 160000
_H = 16
_EPS = 1e-5

_EBLK = 8000
_NBLK = 1000

_NC = 2
_NS = 16
_NW = _NC * _NS
_PERW = _E // _NW
_GCH = 128
_NFULL = _PERW // _GCH
_TAIL = _PERW - _NFULL * _GCH


def _gather_body(tbl_hbm, idx_hbm, out_hbm, idx_v, rows_v, sem):
    wid = lax.axis_index("s") * _NC + lax.axis_index("c")
    base = wid * _PERW
    pltpu.sync_copy(idx_hbm.at[pl.ds(base, _PERW)], idx_v)

    @pl.loop(0, _NFULL)
    def _fire(j):
        pltpu.make_async_copy(
            tbl_hbm.at[idx_v.at[pl.ds(j * _GCH, _GCH)]],
            rows_v.at[pl.ds(j * _GCH, _GCH)], sem).start()

    pltpu.make_async_copy(
        tbl_hbm.at[idx_v.at[pl.ds(_NFULL * _GCH, _TAIL)]],
        rows_v.at[pl.ds(_NFULL * _GCH, _TAIL)], sem).start()
    pltpu.make_async_copy(tbl_hbm.at[pl.ds(0, _PERW)], rows_v, sem).wait()
    pltpu.sync_copy(rows_v, out_hbm.at[pl.ds(base, _PERW)])


def _gather(tbl, idx):
    return pl.kernel(
        _gather_body,
        out_type=jax.ShapeDtypeStruct((_E, _H), jnp.float32),
        mesh=plsc.VectorSubcoreMesh(core_axis_name="c", subcore_axis_name="s"),
        scratch_types=[
            pltpu.VMEM((_PERW,), jnp.int32),
            pltpu.VMEM((_PERW, _H), jnp.float32),
            pltpu.SemaphoreType.DMA,
        ],
        compiler_params=pltpu.CompilerParams(use_tc_tiling_on_sc=False),
    )(tbl, idx)


_CCH = 125
_NCHK = _PERW // _CCH
_NSTRIPE = _N // _NS


def _scatter_body(msg_hbm, dst_hbm, zeros_hbm, out_hbm, idx_v, rows_v, agg_sh):
    c = lax.axis_index("c")
    s = lax.axis_index("s")
    wid = s * _NC + c
    base = wid * _PERW
    pltpu.sync_copy(zeros_hbm.at[pl.ds(s * _NSTRIPE, _NSTRIPE)],
                    agg_sh.at[pl.ds(s * _NSTRIPE, _NSTRIPE)])
    pltpu.sync_copy(dst_hbm.at[wid], idx_v)
    pltpu.sync_copy(msg_hbm.at[pl.ds(base, _PERW)], rows_v)
    plsc.subcore_barrier()

    @pl.loop(0, _NCHK)
    def _sc(j):
        pltpu.sync_copy(rows_v.at[pl.ds(j * _CCH, _CCH)],
                        agg_sh.at[idx_v.at[j]], add=True)

    plsc.subcore_barrier()
    pltpu.sync_copy(agg_sh.at[pl.ds(s * _NSTRIPE, _NSTRIPE)],
                    out_hbm.at[c, pl.ds(s * _NSTRIPE, _NSTRIPE)])


def _scatter(msg, dst3d, zeros_n):
    return pl.kernel(
        _scatter_body,
        out_type=jax.ShapeDtypeStruct((_NC, _N, _H), jnp.float32),
        mesh=plsc.VectorSubcoreMesh(core_axis_name="c", subcore_axis_name="s"),
        scratch_types=[
            pltpu.VMEM((_NCHK, _CCH), jnp.int32),
            pltpu.VMEM((_PERW, _H), jnp.float32),
            pltpu.VMEM_SHARED((_N, _H), jnp.float32),
        ],
        compiler_params=pltpu.CompilerParams(use_tc_tiling_on_sc=False),
    )(msg, dst3d, zeros_n)


def _deg_body(dst_hbm, zeros_hbm, ones_hbm, out_hbm, idx_v, ones_v, deg_sh):
    c = lax.axis_index("c")
    s = lax.axis_index("s")
    wid = s * _NC + c
    pltpu.sync_copy(zeros_hbm.at[pl.ds(s * _NSTRIPE, _NSTRIPE)],
                    deg_sh.at[pl.ds(s * _NSTRIPE, _NSTRIPE)])
    pltpu.sync_copy(dst_hbm.at[wid], idx_v)
    pltpu.sync_copy(ones_hbm, ones_v)
    plsc.subcore_barrier()

    @pl.loop(0, _NCHK)
    def _sc(j):
        pltpu.sync_copy(ones_v, deg_sh.at[idx_v.at[j]], add=True)

    plsc.subcore_barrier()
    pltpu.sync_copy(deg_sh.at[pl.ds(s * _NSTRIPE, _NSTRIPE)],
                    out_hbm.at[c, pl.ds(s * _NSTRIPE, _NSTRIPE)])


def _degrees(dst3d, zeros_n, ones_c):
    return pl.kernel(
        _deg_body,
        out_type=jax.ShapeDtypeStruct((_NC, _N, _H), jnp.float32),
        mesh=plsc.VectorSubcoreMesh(core_axis_name="c", subcore_axis_name="s"),
        scratch_types=[
            pltpu.VMEM((_NCHK, _CCH), jnp.int32),
            pltpu.VMEM((_CCH, _H), jnp.float32),
            pltpu.VMEM_SHARED((_N, _H), jnp.float32),
        ],
        compiler_params=pltpu.CompilerParams(use_tc_tiling_on_sc=False),
    )(dst3d, zeros_n, ones_c)


def _embed_body(x_ref, w_ref, b_ref, s_ref, stat_ref):
    i = pl.program_id(0)
    s = jax.nn.relu(
        jnp.dot(x_ref[...], w_ref[...], preferred_element_type=jnp.float32)
        + b_ref[...])
    s_ref[...] = s

    @pl.when(i == 0)
    def _():
        stat_ref[...] = jnp.zeros_like(stat_ref)

    stat_ref[0:1, :] += jnp.sum(s, axis=0, keepdims=True)
    stat_ref[1:2, :] += jnp.sum(s * s, axis=0, keepdims=True)


def _embed(x, ne_w, ne_b):
    nblk = _N // _NBLK
    s, stat = pl.pallas_call(
        _embed_body,
        grid=(nblk,),
        in_specs=[
            pl.BlockSpec((_NBLK, 128), lambda i: (i, 0)),
            pl.BlockSpec((128, _H), lambda i: (0, 0)),
            pl.BlockSpec((1, _H), lambda i: (0, 0)),
        ],
        out_specs=[
            pl.BlockSpec((_NBLK, _H), lambda i: (i, 0)),
            pl.BlockSpec((8, _H), lambda i: (0, 0)),
        ],
        out_shape=[
            jax.ShapeDtypeStruct((_N, _H), jnp.float32),
            jax.ShapeDtypeStruct((8, _H), jnp.float32),
        ],
    )(x, ne_w, ne_b.reshape(1, _H))
    return s, stat


def _norm_body(s_ref, stat_ref, g_ref, be_ref, h_ref):
    mu = stat_ref[0:1, :] / _N
    var = stat_ref[1:2, :] / _N - mu * mu
    scale = g_ref[...] * lax.rsqrt(var + _EPS)
    shift = be_ref[...] - mu * scale
    h_ref[...] = s_ref[...] * scale + shift


def _normalize(s, stat, gamma, beta):
    nblk = _N // _NBLK
    return pl.pallas_call(
        _norm_body,
        grid=(nblk,),
        in_specs=[
            pl.BlockSpec((_NBLK, _H), lambda i: (i, 0)),
            pl.BlockSpec((8, _H), lambda i: (0, 0)),
            pl.BlockSpec((1, _H), lambda i: (0, 0)),
            pl.BlockSpec((1, _H), lambda i: (0, 0)),
        ],
        out_specs=pl.BlockSpec((_NBLK, _H), lambda i: (i, 0)),
        out_shape=jax.ShapeDtypeStruct((_N, _H), jnp.float32),
    )(s, stat, gamma.reshape(1, _H), beta.reshape(1, _H))


def _msg_body(ea_ref, hs_ref, w1_ref, b1_ref, w2_ref, b2_ref, p_ref, sm_ref,
              out_ref):
    a1 = jax.nn.relu(
        jnp.dot(ea_ref[...], w1_ref[...], preferred_element_type=jnp.float32)
        + b1_ref[...])
    we = (jnp.dot(a1.astype(jnp.bfloat16), w2_ref[...],
                  preferred_element_type=jnp.float32) + b2_ref[...])
    hsrep = jnp.dot(hs_ref[...].astype(jnp.bfloat16), p_ref[...],
                    preferred_element_type=jnp.float32)
    q = (hsrep * we).astype(jnp.bfloat16)
    out_ref[...] = jnp.dot(q, sm_ref[...], preferred_element_type=jnp.float32)


def _messages(ea, hs, w1, b1, w2, b2, pmat, smat):
    nblk = _E // _EBLK
    return pl.pallas_call(
        _msg_body,
        grid=(nblk,),
        in_specs=[
            pl.BlockSpec((_EBLK, _H), lambda i: (i, 0)),
            pl.BlockSpec((_EBLK, _H), lambda i: (i, 0)),
            pl.BlockSpec((_H, 2 * _H), lambda i: (0, 0)),
            pl.BlockSpec((1, 2 * _H), lambda i: (0, 0)),
            pl.BlockSpec((2 * _H, _H * _H), lambda i: (0, 0)),
            pl.BlockSpec((1, _H * _H), lambda i: (0, 0)),
            pl.BlockSpec((_H, _H * _H), lambda i: (0, 0)),
            pl.BlockSpec((_H * _H, _H), lambda i: (0, 0)),
        ],
        out_specs=pl.BlockSpec((_EBLK, _H), lambda i: (i, 0)),
        out_shape=jax.ShapeDtypeStruct((_E, _H), jnp.float32),
    )(ea, hs, w1, b1.reshape(1, -1).astype(jnp.float32),
      w2, b2.reshape(1, -1).astype(jnp.float32), pmat, smat)


def _upd_body(h_ref, a0_ref, a1_ref, dinv_ref, root_ref, bias_ref, out_ref):
    out_ref[...] = jax.nn.relu(
        jnp.dot(h_ref[...], root_ref[...], preferred_element_type=jnp.float32)
        + (a0_ref[...] + a1_ref[...]) * dinv_ref[...] + bias_ref[...])


def _update(h, aggp, dinv, root, bias):
    nblk = _N // _NBLK
    return pl.pallas_call(
        _upd_body,
        grid=(nblk,),
        in_specs=[
            pl.BlockSpec((_NBLK, _H), lambda i: (i, 0)),
            pl.BlockSpec((_NBLK, _H), lambda i: (i, 0)),
            pl.BlockSpec((_NBLK, _H), lambda i: (i, 0)),
            pl.BlockSpec((_NBLK, _H), lambda i: (i, 0)),
            pl.BlockSpec((_H, _H), lambda i: (0, 0)),
            pl.BlockSpec((1, _H), lambda i: (0, 0)),
        ],
        out_specs=pl.BlockSpec((_NBLK, _H), lambda i: (i, 0)),
        out_shape=jax.ShapeDtypeStruct((_N, _H), jnp.float32),
    )(h, aggp[0], aggp[1], dinv, root, bias.reshape(1, _H))


def _dinv_body(d0_ref, d1_ref, out_ref):
    out_ref[...] = 1.0 / jnp.clip(d0_ref[...] + d1_ref[...], 1.0, None)


def _dinv(degp):
    nblk = _N // _NBLK
    return pl.pallas_call(
        _dinv_body,
        grid=(nblk,),
        in_specs=[
            pl.BlockSpec((_NBLK, _H), lambda i: (i, 0)),
            pl.BlockSpec((_NBLK, _H), lambda i: (i, 0)),
        ],
        out_specs=pl.BlockSpec((_NBLK, _H), lambda i: (i, 0)),
        out_shape=jax.ShapeDtypeStruct((_N, _H), jnp.float32),
    )(degp[0], degp[1])


def kernel(x, edge_attr, ne_w, ne_b, bn_gamma, bn_beta, conv_w1, conv_b1,
           conv_w2, conv_b2, conv_root, conv_bias, wk_w1, wk_b1, wk_w2, wk_b2,
           uj_w1, uj_b1, uj_w2, uj_b2, zk_w1, zk_b1, zk_w2, zk_b2, edge_index,
           batch):
    L = conv_w1.shape[0]
    B = 8
    src = edge_index[0]
    dst = edge_index[1]

    eye = jnp.eye(_H, dtype=jnp.bfloat16)
    pmat = jnp.repeat(eye, _H, axis=1)
    smat = jnp.tile(eye, (_H, 1))
    ea_bf = edge_attr.astype(jnp.bfloat16)

    dst3d = dst.reshape(_NW, _NCHK, _CCH)
    zeros_n = jnp.zeros((_N, _H), jnp.float32)
    ones_c = jnp.ones((_CCH, _H), jnp.float32)

    degp = _degrees(dst3d, zeros_n, ones_c)
    dinv = _dinv(degp)

    s, stat = _embed(x, ne_w, ne_b)
    h = _normalize(s, stat, bn_gamma, bn_beta)

    for l in range(L):
        hs = _gather(h, src)
        msg = _messages(ea_bf, hs, conv_w1[l].astype(jnp.bfloat16),
                        conv_b1[l], conv_w2[l].astype(jnp.bfloat16),
                        conv_b2[l], pmat, smat)
        aggp = _scatter(msg, dst3d, zeros_n)
        h = _update(h, aggp, dinv, conv_root[l], conv_bias[l])

    gcnt = jnp.clip(
        jax.ops.segment_sum(jnp.ones((_N,), jnp.float32), batch,
                            num_segments=B), 1.0, None)[:, None]
    hg = jax.ops.segment_sum(h, batch, num_segments=B) / gcnt
    wk = (jax.nn.relu(hg @ wk_w1 + wk_b1) @ wk_w2 + wk_b2).squeeze(-1)
    uj = jax.nn.relu(hg @ uj_w1 + uj_b1) @ uj_w2 + uj_b2
    zk = jax.nn.relu(hg @ zk_w1 + zk_b1) @ zk_w2 + zk_b2
    return (wk, uj, zk)


# speedup vs baseline: 4.0047x; 1.0669x over previous
import functools

import jax
import jax.numpy as jnp
from jax import lax
from jax.experimental import pallas as pl
from jax.experimental.pallas import tpu as pltpu
from jax.experimental.pallas import tpu_sc as plsc

_N = 10000
_E = 160000
_H = 16
_EPS = 1e-5

_EBLK = 8000
_NBLK = 1000

_NC = 2
_NS = 16
_NW = _NC * _NS
_PERW = _E // _NW
_GCH = 128
_NFULL = _PERW // _GCH
_TAIL = _PERW - _NFULL * _GCH


def _gather_body(tbl_hbm, idx_hbm, out_hbm, idx_v, rows_v, sem):
    wid = lax.axis_index("s") * _NC + lax.axis_index("c")
    base = wid * _PERW
    pltpu.sync_copy(idx_hbm.at[pl.ds(base, _PERW)], idx_v)

    @pl.loop(0, _NFULL)
    def _fire(j):
        pltpu.make_async_copy(
            tbl_hbm.at[idx_v.at[pl.ds(j * _GCH, _GCH)]],
            rows_v.at[pl.ds(j * _GCH, _GCH)], sem).start()

    pltpu.make_async_copy(
        tbl_hbm.at[idx_v.at[pl.ds(_NFULL * _GCH, _TAIL)]],
        rows_v.at[pl.ds(_NFULL * _GCH, _TAIL)], sem).start()
    pltpu.make_async_copy(tbl_hbm.at[pl.ds(0, _PERW)], rows_v, sem).wait()
    pltpu.sync_copy(rows_v, out_hbm.at[pl.ds(base, _PERW)])


def _gather(tbl, idx):
    return pl.kernel(
        _gather_body,
        out_type=jax.ShapeDtypeStruct((_E, _H), jnp.float32),
        mesh=plsc.VectorSubcoreMesh(core_axis_name="c", subcore_axis_name="s"),
        scratch_types=[
            pltpu.VMEM((_PERW,), jnp.int32),
            pltpu.VMEM((_PERW, _H), jnp.float32),
            pltpu.SemaphoreType.DMA,
        ],
        compiler_params=pltpu.CompilerParams(use_tc_tiling_on_sc=False),
    )(tbl, idx)


_CCH = 125
_NCHK = _PERW // _CCH
_NSTRIPE = _N // _NS


def _scatter_body(msg_hbm, dst_hbm, zeros_hbm, out_hbm, idx_v, rows_v, agg_sh):
    c = lax.axis_index("c")
    s = lax.axis_index("s")
    wid = s * _NC + c
    base = wid * _PERW
    pltpu.sync_copy(zeros_hbm.at[pl.ds(s * _NSTRIPE, _NSTRIPE)],
                    agg_sh.at[pl.ds(s * _NSTRIPE, _NSTRIPE)])
    pltpu.sync_copy(dst_hbm.at[wid], idx_v)
    pltpu.sync_copy(msg_hbm.at[pl.ds(base, _PERW)], rows_v)
    plsc.subcore_barrier()

    @pl.loop(0, _NCHK)
    def _sc(j):
        pltpu.sync_copy(rows_v.at[pl.ds(j * _CCH, _CCH)],
                        agg_sh.at[idx_v.at[j]], add=True)

    plsc.subcore_barrier()
    pltpu.sync_copy(agg_sh.at[pl.ds(s * _NSTRIPE, _NSTRIPE)],
                    out_hbm.at[c, pl.ds(s * _NSTRIPE, _NSTRIPE)])


def _scatter(msg, dst3d, zeros_n):
    return pl.kernel(
        _scatter_body,
        out_type=jax.ShapeDtypeStruct((_NC, _N, _H), jnp.float32),
        mesh=plsc.VectorSubcoreMesh(core_axis_name="c", subcore_axis_name="s"),
        scratch_types=[
            pltpu.VMEM((_NCHK, _CCH), jnp.int32),
            pltpu.VMEM((_PERW, _H), jnp.float32),
            pltpu.VMEM_SHARED((_N, _H), jnp.float32),
        ],
        compiler_params=pltpu.CompilerParams(use_tc_tiling_on_sc=False),
    )(msg, dst3d, zeros_n)


def _deg_body(dst_hbm, zeros_hbm, ones_hbm, out_hbm, idx_v, ones_v, deg_sh):
    c = lax.axis_index("c")
    s = lax.axis_index("s")
    wid = s * _NC + c
    pltpu.sync_copy(zeros_hbm.at[pl.ds(s * _NSTRIPE, _NSTRIPE)],
                    deg_sh.at[pl.ds(s * _NSTRIPE, _NSTRIPE)])
    pltpu.sync_copy(dst_hbm.at[wid], idx_v)
    pltpu.sync_copy(ones_hbm, ones_v)
    plsc.subcore_barrier()

    @pl.loop(0, _NCHK)
    def _sc(j):
        pltpu.sync_copy(ones_v, deg_sh.at[idx_v.at[j]], add=True)

    plsc.subcore_barrier()
    pltpu.sync_copy(deg_sh.at[pl.ds(s * _NSTRIPE, _NSTRIPE)],
                    out_hbm.at[c, pl.ds(s * _NSTRIPE, _NSTRIPE)])


def _degrees(dst3d, zeros_n, ones_c):
    return pl.kernel(
        _deg_body,
        out_type=jax.ShapeDtypeStruct((_NC, _N, _H), jnp.float32),
        mesh=plsc.VectorSubcoreMesh(core_axis_name="c", subcore_axis_name="s"),
        scratch_types=[
            pltpu.VMEM((_NCHK, _CCH), jnp.int32),
            pltpu.VMEM((_CCH, _H), jnp.float32),
            pltpu.VMEM_SHARED((_N, _H), jnp.float32),
        ],
        compiler_params=pltpu.CompilerParams(use_tc_tiling_on_sc=False),
    )(dst3d, zeros_n, ones_c)


def _embed_body(x_ref, w_ref, b_ref, s_ref, stat_ref):
    i = pl.program_id(0)
    s = jax.nn.relu(
        jnp.dot(x_ref[...], w_ref[...], preferred_element_type=jnp.float32)
        + b_ref[...])
    s_ref[...] = s

    @pl.when(i == 0)
    def _():
        stat_ref[...] = jnp.zeros_like(stat_ref)

    stat_ref[0:1, :] += jnp.sum(s, axis=0, keepdims=True)
    stat_ref[1:2, :] += jnp.sum(s * s, axis=0, keepdims=True)


def _embed(x, ne_w, ne_b):
    nblk = _N // _NBLK
    s, stat = pl.pallas_call(
        _embed_body,
        grid=(nblk,),
        in_specs=[
            pl.BlockSpec((_NBLK, 128), lambda i: (i, 0)),
            pl.BlockSpec((128, _H), lambda i: (0, 0)),
            pl.BlockSpec((1, _H), lambda i: (0, 0)),
        ],
        out_specs=[
            pl.BlockSpec((_NBLK, _H), lambda i: (i, 0)),
            pl.BlockSpec((8, _H), lambda i: (0, 0)),
        ],
        out_shape=[
            jax.ShapeDtypeStruct((_N, _H), jnp.float32),
            jax.ShapeDtypeStruct((8, _H), jnp.float32),
        ],
    )(x, ne_w, ne_b.reshape(1, _H))
    return s, stat


def _norm_body(s_ref, stat_ref, g_ref, be_ref, h_ref):
    mu = stat_ref[0:1, :] / _N
    var = stat_ref[1:2, :] / _N - mu * mu
    scale = g_ref[...] * lax.rsqrt(var + _EPS)
    shift = be_ref[...] - mu * scale
    h_ref[...] = s_ref[...] * scale + shift


def _normalize(s, stat, gamma, beta):
    nblk = _N // _NBLK
    return pl.pallas_call(
        _norm_body,
        grid=(nblk,),
        in_specs=[
            pl.BlockSpec((_NBLK, _H), lambda i: (i, 0)),
            pl.BlockSpec((8, _H), lambda i: (0, 0)),
            pl.BlockSpec((1, _H), lambda i: (0, 0)),
            pl.BlockSpec((1, _H), lambda i: (0, 0)),
        ],
        out_specs=pl.BlockSpec((_NBLK, _H), lambda i: (i, 0)),
        out_shape=jax.ShapeDtypeStruct((_N, _H), jnp.float32),
    )(s, stat, gamma.reshape(1, _H), beta.reshape(1, _H))


def _msg_body(ea_ref, hs_ref, w1_ref, b1_ref, w2_ref, b2_ref, p_ref, sm_ref,
              out_ref):
    a1 = jax.nn.relu(
        jnp.dot(ea_ref[...], w1_ref[...], preferred_element_type=jnp.float32)
        + b1_ref[...])
    we = (jnp.dot(a1.astype(jnp.bfloat16), w2_ref[...],
                  preferred_element_type=jnp.float32) + b2_ref[...])
    hsrep = jnp.dot(hs_ref[...].astype(jnp.bfloat16), p_ref[...],
                    preferred_element_type=jnp.float32)
    q = (hsrep * we).astype(jnp.bfloat16)
    out_ref[...] = jnp.dot(q, sm_ref[...], preferred_element_type=jnp.float32)


def _messages(ea, hs, w1, b1, w2, b2, pmat, smat):
    nblk = _E // _EBLK
    return pl.pallas_call(
        _msg_body,
        grid=(nblk,),
        in_specs=[
            pl.BlockSpec((_EBLK, _H), lambda i: (i, 0)),
            pl.BlockSpec((_EBLK, _H), lambda i: (i, 0)),
            pl.BlockSpec((_H, 2 * _H), lambda i: (0, 0)),
            pl.BlockSpec((1, 2 * _H), lambda i: (0, 0)),
            pl.BlockSpec((2 * _H, _H * _H), lambda i: (0, 0)),
            pl.BlockSpec((1, _H * _H), lambda i: (0, 0)),
            pl.BlockSpec((_H, _H * _H), lambda i: (0, 0)),
            pl.BlockSpec((_H * _H, _H), lambda i: (0, 0)),
        ],
        out_specs=pl.BlockSpec((_EBLK, _H), lambda i: (i, 0)),
        out_shape=jax.ShapeDtypeStruct((_E, _H), jnp.float32),
    )(ea, hs, w1, b1.reshape(1, -1).astype(jnp.float32),
      w2, b2.reshape(1, -1).astype(jnp.float32), pmat, smat)


def _upd_body(h_ref, a0_ref, a1_ref, dinv_ref, root_ref, bias_ref, out_ref):
    out_ref[...] = jax.nn.relu(
        jnp.dot(h_ref[...], root_ref[...], preferred_element_type=jnp.float32)
        + (a0_ref[...] + a1_ref[...]) * dinv_ref[...] + bias_ref[...])


def _update(h, aggp, dinv, root, bias):
    nblk = _N // _NBLK
    return pl.pallas_call(
        _upd_body,
        grid=(nblk,),
        in_specs=[
            pl.BlockSpec((_NBLK, _H), lambda i: (i, 0)),
            pl.BlockSpec((_NBLK, _H), lambda i: (i, 0)),
            pl.BlockSpec((_NBLK, _H), lambda i: (i, 0)),
            pl.BlockSpec((_NBLK, _H), lambda i: (i, 0)),
            pl.BlockSpec((_H, _H), lambda i: (0, 0)),
            pl.BlockSpec((1, _H), lambda i: (0, 0)),
        ],
        out_specs=pl.BlockSpec((_NBLK, _H), lambda i: (i, 0)),
        out_shape=jax.ShapeDtypeStruct((_N, _H), jnp.float32),
    )(h, aggp[0], aggp[1], dinv, root, bias.reshape(1, _H))


def _dinv_body(d0_ref, d1_ref, out_ref):
    out_ref[...] = 1.0 / jnp.clip(d0_ref[...] + d1_ref[...], 1.0, None)


def _dinv(degp):
    nblk = _N // _NBLK
    return pl.pallas_call(
        _dinv_body,
        grid=(nblk,),
        in_specs=[
            pl.BlockSpec((_NBLK, _H), lambda i: (i, 0)),
            pl.BlockSpec((_NBLK, _H), lambda i: (i, 0)),
        ],
        out_specs=pl.BlockSpec((_NBLK, _H), lambda i: (i, 0)),
        out_shape=jax.ShapeDtypeStruct((_N, _H), jnp.float32),
    )(degp[0], degp[1])


def _pool_body(h_ref, b_ref, out_ref):
    i = pl.program_id(0)

    @pl.when(i == 0)
    def _():
        out_ref[...] = jnp.zeros_like(out_ref)

    batch_row = b_ref[...].reshape(1, _NBLK)
    gids = lax.broadcasted_iota(jnp.int32, (8, _NBLK), 0)
    oh = (gids == batch_row).astype(jnp.float32)
    hext = jnp.concatenate(
        [h_ref[...], jnp.ones((_NBLK, _H), jnp.float32)], axis=1)
    out_ref[...] += jnp.dot(oh, hext, preferred_element_type=jnp.float32)


def _pool(h, batch3d):
    nblk = _N // _NBLK
    return pl.pallas_call(
        _pool_body,
        grid=(nblk,),
        in_specs=[
            pl.BlockSpec((_NBLK, _H), lambda i: (i, 0)),
            pl.BlockSpec((1, 1, _NBLK), lambda i: (i, 0, 0)),
        ],
        out_specs=pl.BlockSpec((8, 2 * _H), lambda i: (0, 0)),
        out_shape=jax.ShapeDtypeStruct((8, 2 * _H), jnp.float32),
    )(h, batch3d)


def _heads_body(p_ref, ww1, wb1, ww2, wb2, uw1, ub1, uw2, ub2, zw1, zb1, zw2,
                zb2, wk_ref, uj_ref, zk_ref):
    pool = p_ref[...]
    cnt = jnp.clip(pool[:, _H:], 1.0, None)
    hg = pool[:, :_H] / cnt

    def mlp(w1, b1, w2, b2, out):
        t = jax.nn.relu(
            jnp.dot(hg, w1[...], preferred_element_type=jnp.float32) + b1[...])
        out[...] = (jnp.dot(t, w2[...], preferred_element_type=jnp.float32)
                    + b2[...])

    mlp(ww1, wb1, ww2, wb2, wk_ref)
    mlp(uw1, ub1, uw2, ub2, uj_ref)
    mlp(zw1, zb1, zw2, zb2, zk_ref)


def _heads(pool, wk_w1, wk_b1, wk_w2p, wk_b2p, uj_w1, uj_b1, uj_w2, uj_b2,
           zk_w1, zk_b1, zk_w2, zk_b2):
    return pl.pallas_call(
        _heads_body,
        out_shape=[
            jax.ShapeDtypeStruct((8, _H), jnp.float32),
            jax.ShapeDtypeStruct((8, 64), jnp.float32),
            jax.ShapeDtypeStruct((8, 64), jnp.float32),
        ],
    )(pool, wk_w1, wk_b1.reshape(1, _H), wk_w2p, wk_b2p.reshape(1, _H),
      uj_w1, uj_b1.reshape(1, _H), uj_w2, uj_b2.reshape(1, 64),
      zk_w1, zk_b1.reshape(1, _H), zk_w2, zk_b2.reshape(1, 64))


def kernel(x, edge_attr, ne_w, ne_b, bn_gamma, bn_beta, conv_w1, conv_b1,
           conv_w2, conv_b2, conv_root, conv_bias, wk_w1, wk_b1, wk_w2, wk_b2,
           uj_w1, uj_b1, uj_w2, uj_b2, zk_w1, zk_b1, zk_w2, zk_b2, edge_index,
           batch):
    L = conv_w1.shape[0]
    B = 8
    src = edge_index[0]
    dst = edge_index[1]

    eye = jnp.eye(_H, dtype=jnp.bfloat16)
    pmat = jnp.repeat(eye, _H, axis=1)
    smat = jnp.tile(eye, (_H, 1))
    ea_bf = edge_attr.astype(jnp.bfloat16)

    dst3d = dst.reshape(_NW, _NCHK, _CCH)
    zeros_n = jnp.zeros((_N, _H), jnp.float32)
    ones_c = jnp.ones((_CCH, _H), jnp.float32)

    degp = _degrees(dst3d, zeros_n, ones_c)
    dinv = _dinv(degp)

    s, stat = _embed(x, ne_w, ne_b)
    h = _normalize(s, stat, bn_gamma, bn_beta)

    for l in range(L):
        hs = _gather(h, src)
        msg = _messages(ea_bf, hs, conv_w1[l].astype(jnp.bfloat16),
                        conv_b1[l], conv_w2[l].astype(jnp.bfloat16),
                        conv_b2[l], pmat, smat)
        aggp = _scatter(msg, dst3d, zeros_n)
        h = _update(h, aggp, dinv, conv_root[l], conv_bias[l])

    batch3d = batch.reshape(_N // _NBLK, 1, _NBLK)
    pool = _pool(h, batch3d)
    wk_w2p = jnp.pad(wk_w2, ((0, 0), (0, _H - 1)))
    wk_b2p = jnp.pad(wk_b2, (0, _H - 1))
    wk_full, uj, zk = _heads(pool, wk_w1, wk_b1, wk_w2p, wk_b2p, uj_w1, uj_b1,
                             uj_w2, uj_b2, zk_w1, zk_b1, zk_w2, zk_b2)
    return (wk_full[:, 0], uj, zk)
